# baseline (speedup 1.0000x reference)
import jax
import jax.numpy as jnp
from jax.experimental import pallas as pl
from jax.experimental.pallas import tpu as pltpu

_BIG = 1e11
_NSUM = 8


def _body(func_ref, state_ref, out_ref, cnt_ref):
    g = func_ref.shape[0]
    for i in range(g):
        f = func_ref[i]
        s = f.shape[-1]
        lab = jax.lax.broadcasted_iota(jnp.int32, (_NSUM, s), 0)
        fb = jnp.broadcast_to(f, (_NSUM, s))
        eq = jnp.where(fb == lab + 1, 1.0, 0.0)
        nonpad = jnp.where(fb != 0, 1.0, 0.0)
        maskf = jnp.where(lab == 5, nonpad, eq)
        sums = jax.lax.dot_general(
            maskf, state_ref[i],
            dimension_numbers=(((1,), (0,)), ((), ())),
            preferred_element_type=jnp.float32,
            precision=jax.lax.Precision.HIGHEST,
        )
        counts = jnp.sum(maskf, axis=1, keepdims=True)
        denom = jnp.where(counts > 0, counts, jnp.float32(_BIG))
        out_ref[i] = sums / denom
        cnt_ref[i] = jnp.broadcast_to(counts, (_NSUM, 128))


def _segmean(state, func3, rows_per_step):
    r, s, h = state.shape
    g = rows_per_step
    assert r % g == 0
    out_shape = (
        jax.ShapeDtypeStruct((r, _NSUM, h), jnp.float32),
        jax.ShapeDtypeStruct((r, _NSUM, 128), jnp.float32),
    )
    return pl.pallas_call(
        _body,
        grid=(r // g,),
        in_specs=[
            pl.BlockSpec((g, 1, s), lambda i: (i, 0, 0)),
            pl.BlockSpec((g, s, h), lambda i: (i, 0, 0)),
        ],
        out_specs=(
            pl.BlockSpec((g, _NSUM, h), lambda i: (i, 0, 0)),
            pl.BlockSpec((g, _NSUM, 128), lambda i: (i, 0, 0)),
        ),
        out_shape=out_shape,
        compiler_params=pltpu.CompilerParams(
            dimension_semantics=("parallel",),
        ),
        name="segmean",
    )(func3, state)


def kernel(tarsent_state, tar_func, refsent_state, ref_func):
    b, ts, h = tarsent_state.shape
    _, d, rs, _ = refsent_state.shape

    tar_out, tar_cnt = _segmean(tarsent_state, tar_func.reshape(b, 1, ts), 1)
    ref_out, ref_cnt = _segmean(
        refsent_state.reshape(b * d, rs, h),
        ref_func.reshape(b * d, 1, rs), 4)

    tar_counts = tar_cnt[:, :, 0]
    tar_aug = tar_out[:, :5, :]
    tar_aug_mask = tar_counts[:, :5] > 0
    tarpaper = tar_out[:, 5, :]
    tar_mask2 = tar_counts[:, 5] > 0

    ref_out = ref_out.reshape(b, d, _NSUM, h)
    ref_counts = ref_cnt[:, :, 0].reshape(b, d, _NSUM)
    ref_aug = ref_out[:, :, :5, :]
    ref_aug_mask = ref_counts[:, :, :5] > 0
    refpaper = ref_out[:, :, 5, :]
    ref_mask2 = ref_counts[:, :, 5] > 0

    return (tar_aug, tar_aug_mask, ref_aug, ref_aug_mask,
            tarpaper, tar_mask2, refpaper, ref_mask2)


# speedup vs baseline: 1.5665x; 1.5665x over previous
import jax
import jax.numpy as jnp
from jax.experimental import pallas as pl
from jax.experimental.pallas import tpu as pltpu

_BIG = 1e11
_NSUM = 8


def _body(func_ref, state_ref, out_ref, cnt_ref):
    g = func_ref.shape[0]
    for i in range(g):
        f = func_ref[i]
        s = f.shape[-1]
        lab = jax.lax.broadcasted_iota(jnp.int32, (_NSUM, s), 0)
        fb = jnp.broadcast_to(f, (_NSUM, s))
        eq = jnp.where(fb == lab + 1, 1.0, 0.0)
        nonpad = jnp.where(fb != 0, 1.0, 0.0)
        maskf = jnp.where(lab == 5, nonpad, eq)
        sums = jax.lax.dot_general(
            maskf, state_ref[i],
            dimension_numbers=(((1,), (0,)), ((), ())),
            preferred_element_type=jnp.float32,
        )
        counts = jnp.sum(maskf, axis=1, keepdims=True)
        denom = jnp.where(counts > 0, counts, jnp.float32(_BIG))
        out_ref[i] = sums / denom
        cnt_ref[i] = jnp.broadcast_to(counts, (_NSUM, 128))


def _segmean(state, func3, rows_per_step):
    r, s, h = state.shape
    g = rows_per_step
    assert r % g == 0
    out_shape = (
        jax.ShapeDtypeStruct((r, _NSUM, h), jnp.float32),
        jax.ShapeDtypeStruct((r, _NSUM, 128), jnp.float32),
    )
    return pl.pallas_call(
        _body,
        grid=(r // g,),
        in_specs=[
            pl.BlockSpec((g, 1, s), lambda i: (i, 0, 0)),
            pl.BlockSpec((g, s, h), lambda i: (i, 0, 0)),
        ],
        out_specs=(
            pl.BlockSpec((g, _NSUM, h), lambda i: (i, 0, 0)),
            pl.BlockSpec((g, _NSUM, 128), lambda i: (i, 0, 0)),
        ),
        out_shape=out_shape,
        compiler_params=pltpu.CompilerParams(
            dimension_semantics=("parallel",),
        ),
        name="segmean",
    )(func3, state)


def kernel(tarsent_state, tar_func, refsent_state, ref_func):
    b, ts, h = tarsent_state.shape
    _, d, rs, _ = refsent_state.shape

    tar_out, tar_cnt = _segmean(tarsent_state, tar_func.reshape(b, 1, ts), 1)
    ref_out, ref_cnt = _segmean(
        refsent_state.reshape(b * d, rs, h),
        ref_func.reshape(b * d, 1, rs), 8)

    tar_counts = tar_cnt[:, :, 0]
    tar_aug = tar_out[:, :5, :]
    tar_aug_mask = tar_counts[:, :5] > 0
    tarpaper = tar_out[:, 5, :]
    tar_mask2 = tar_counts[:, 5] > 0

    ref_out = ref_out.reshape(b, d, _NSUM, h)
    ref_counts = ref_cnt[:, :, 0].reshape(b, d, _NSUM)
    ref_aug = ref_out[:, :, :5, :]
    ref_aug_mask = ref_counts[:, :, :5] > 0
    refpaper = ref_out[:, :, 5, :]
    ref_mask2 = ref_counts[:, :, 5] > 0

    return (tar_aug, tar_aug_mask, ref_aug, ref_aug_mask,
            tarpaper, tar_mask2, refpaper, ref_mask2)
